# baseline (speedup 1.0000x reference)
import functools

import jax
import jax.numpy as jnp
from jax import lax
from jax.experimental import pallas as pl
from jax.experimental.pallas import tpu as pltpu
from jax.experimental.pallas import tpu_sc as plsc

_K = 8192
_D = 32
_N = 6272

_M_TILE = 784
_K_TILE = 1024
_MT = _N // _M_TILE
_KT = _K // _K_TILE


def _conv(x, w, b, stride):
    y = jax.lax.conv_general_dilated(
        x, w, window_strides=(stride, stride), padding=((1, 1), (1, 1)),
        dimension_numbers=("NCHW", "OIHW", "NCHW"))
    return y + b.reshape(1, -1, 1, 1)


def _conv_transpose(x, w, b):
    wf = jnp.flip(w, axis=(2, 3)).transpose(1, 0, 2, 3)
    y = jax.lax.conv_general_dilated(
        x, wf, window_strides=(1, 1), padding=((2, 2), (2, 2)),
        lhs_dilation=(2, 2), dimension_numbers=("NCHW", "OIHW", "NCHW"))
    return y + b.reshape(1, -1, 1, 1)


def _vq_argmin_body(flat_ref, fsq_ref, cb_ref, csq_ref, q_ref, loss_ref,
                    rmin_ref, acc_ref):
    m = pl.program_id(0)
    k = pl.program_id(1)
    mm = lax.dot_general(flat_ref[...], cb_ref[...],
                         (((1,), (1,)), ((), ())),
                         preferred_element_type=jnp.float32)
    dist = (fsq_ref[...] + csq_ref[...]) - 2.0 * mm
    lmin = jnp.min(dist, axis=1, keepdims=True)
    larg = (jnp.argmin(dist, axis=1).astype(jnp.int32).reshape(_M_TILE, 1)
            + k * _K_TILE)

    @pl.when(k == 0)
    def _init():
        rmin_ref[...] = lmin
        q_ref[...] = larg

    @pl.when(k > 0)
    def _update():
        prev = rmin_ref[...]
        better = lmin < prev
        rmin_ref[...] = jnp.where(better, lmin, prev)
        q_ref[...] = jnp.where(better, larg, q_ref[...])

    @pl.when(k == _KT - 1)
    def _finish():
        s = jnp.sum(rmin_ref[...])
        prev = jnp.where(m == 0, 0.0, acc_ref[0, 0])
        acc_ref[0, 0] = prev + s

        @pl.when(m == _MT - 1)
        def _emit():
            loss_ref[0, 0] = acc_ref[0, 0]


def _vq_argmin(flat, flat_sq, codebook, cb_sq):
    q, loss_sum = pl.pallas_call(
        _vq_argmin_body,
        grid=(_MT, _KT),
        in_specs=[
            pl.BlockSpec((_M_TILE, _D), lambda m, k: (m, 0)),
            pl.BlockSpec((_M_TILE, 1), lambda m, k: (m, 0)),
            pl.BlockSpec((_K_TILE, _D), lambda m, k: (k, 0)),
            pl.BlockSpec((1, _K_TILE), lambda m, k: (0, k)),
        ],
        out_specs=[
            pl.BlockSpec((_M_TILE, 1), lambda m, k: (m, 0)),
            pl.BlockSpec(memory_space=pltpu.SMEM),
        ],
        out_shape=[
            jax.ShapeDtypeStruct((_N, 1), jnp.int32),
            jax.ShapeDtypeStruct((1, 1), jnp.float32),
        ],
        scratch_shapes=[
            pltpu.VMEM((_M_TILE, 1), jnp.float32),
            pltpu.SMEM((1, 1), jnp.float32),
        ],
    )(flat, flat_sq, codebook, cb_sq)
    return q.reshape(_N), loss_sum[0, 0]


_CHUNK = 112
_CHUNKS_PER_W = 2
_B_PER_W = _CHUNK * _CHUNKS_PER_W
_NW = 32
_N_PAD = _B_PER_W * _NW


def _sc_gather_body(table_hbm, idx_hbm, out_hbm, idx_v, rows_v, sem):
    wid = lax.axis_index("s") * 2 + lax.axis_index("c")
    pltpu.sync_copy(idx_hbm.at[pl.ds(wid * _CHUNKS_PER_W, _CHUNKS_PER_W)],
                    idx_v)
    for j in range(_CHUNKS_PER_W):
        pltpu.async_copy(table_hbm.at[idx_v.at[j]], rows_v.at[j], sem).wait()
    pltpu.sync_copy(rows_v,
                    out_hbm.at[pl.ds(wid * _CHUNKS_PER_W, _CHUNKS_PER_W)])


def _sc_gather(codebook, q):
    q_pad = jnp.concatenate(
        [q, jnp.zeros((_N_PAD - _N,), dtype=jnp.int32)]).reshape(
            _NW * _CHUNKS_PER_W, _CHUNK)
    mesh = plsc.VectorSubcoreMesh(core_axis_name="c", subcore_axis_name="s")
    gathered = pl.kernel(
        _sc_gather_body,
        mesh=mesh,
        out_type=jax.ShapeDtypeStruct((_NW * _CHUNKS_PER_W, _CHUNK, _D),
                                      jnp.float32),
        scratch_types=[
            pltpu.VMEM((_CHUNKS_PER_W, _CHUNK), jnp.int32),
            pltpu.VMEM((_CHUNKS_PER_W, _CHUNK, _D), jnp.float32),
            pltpu.SemaphoreType.DMA,
        ],
        compiler_params=pltpu.CompilerParams(use_tc_tiling_on_sc=False),
    )(codebook, q_pad)
    return gathered.reshape(_N_PAD, _D)[:_N]


def kernel(imgs, enc_w1, enc_b1, enc_w2, enc_b2, codebook, dec_w1, dec_b1,
           dec_w2, dec_b2):
    z_e = jax.nn.relu(_conv(imgs, enc_w1, enc_b1, 2))
    z_e = jax.nn.relu(_conv(z_e, enc_w2, enc_b2, 2))
    z = jnp.transpose(z_e, (0, 2, 3, 1))
    z_shape = z.shape
    flat = z.reshape(-1, _D)

    flat_sq = jnp.sum(flat ** 2, axis=1, keepdims=True)
    cb_sq = jnp.sum(codebook ** 2, axis=1).reshape(1, _K)

    q, loss_sum = _vq_argmin(flat, flat_sq, codebook, cb_sq)
    z_q_flat = _sc_gather(codebook, q)

    vq_loss = loss_sum * (2.0 / (_N * _D))

    z_q = z + (z_q_flat.reshape(z_shape) - z)
    encoded = jnp.transpose(z_q, (0, 3, 1, 2))
    d = jax.nn.relu(_conv_transpose(encoded, dec_w1, dec_b1))
    decoded = jax.nn.relu(_conv_transpose(d, dec_w2, dec_b2))
    return encoded, decoded, vq_loss
